# baseline (speedup 1.0000x reference)
import functools
import math

import jax
import jax.numpy as jnp
from jax import lax
from jax.experimental import pallas as pl
from jax.experimental.pallas import tpu as pltpu
from jax.experimental.pallas import tpu_sc as plsc

_B, _S, _V = 4, 2048, 16384
_SMOOTHING = 0.1
_CONF = 1.0 - _SMOOTHING
_R = _B * _S
_FILL = _SMOOTHING / (_V - 1)

_ROW_CONST = (_V - 1) * _FILL * math.log(_FILL) + _CONF * math.log(_CONF)
_TOTAL_CONST = _R * _ROW_CONST

_TOTAL_VECS = _R * _V // (8 * 128)
_VECS_PER_BLK = 4096
_NBLK = _TOTAL_VECS // _VECS_PER_BLK


def _tc_sum_body(x_ref, out_ref, acc_ref):
    i = pl.program_id(0)

    @pl.when(i == 0)
    def _init():
        acc_ref[...] = jnp.zeros_like(acc_ref)

    acc_ref[...] += jnp.sum(x_ref[...], axis=0)

    @pl.when(i == _NBLK - 1)
    def _fini():
        out_ref[0, 0] = jnp.sum(acc_ref[...])


_tc_sum = pl.pallas_call(
    _tc_sum_body,
    grid=(_NBLK,),
    in_specs=[pl.BlockSpec((_VECS_PER_BLK, 8, 128), lambda i: (i, 0, 0))],
    out_specs=pl.BlockSpec(memory_space=pltpu.SMEM),
    out_shape=jax.ShapeDtypeStruct((1, 1), jnp.float32),
    scratch_shapes=[pltpu.VMEM((8, 128), jnp.float32)],
)

_NC, _NS, _L = 2, 16, 16
_NW = _NC * _NS
_RPW = _R // _NW
_CHUNK = 128
_NCHUNK = _RPW // _CHUNK


def _sc_gather_body(x_hbm, tgt_hbm, out_hbm, tgt_v, idx_v, val_v, acc_v, sem):
    cid = lax.axis_index("c")
    sid = lax.axis_index("s")
    wid = sid * _NC + cid
    base = wid * _RPW

    pltpu.sync_copy(tgt_hbm.at[pl.ds(base, _RPW)], tgt_v)

    acc = jnp.zeros((_L,), jnp.float32)
    for h in range(_NCHUNK):
        for j in range(_CHUNK // _L):
            r0 = h * _CHUNK + j * _L
            rows = (base + r0) + lax.iota(jnp.int32, _L)
            idx_v[pl.ds(j * _L, _L)] = rows * _V + tgt_v[pl.ds(r0, _L)]
        pltpu.async_copy(x_hbm.at[idx_v], val_v, sem).wait()
        for j in range(_CHUNK // _L):
            acc = acc + val_v[pl.ds(j * _L, _L)]

    acc_v[...] = acc
    pltpu.sync_copy(acc_v, out_hbm.at[wid])


_sc_gather = functools.partial(
    pl.kernel,
    mesh=plsc.VectorSubcoreMesh(core_axis_name="c", subcore_axis_name="s"),
    out_type=jax.ShapeDtypeStruct((_NW, _L), jnp.float32),
    scratch_types=[
        pltpu.VMEM((_RPW,), jnp.int32),
        pltpu.VMEM((_CHUNK,), jnp.int32),
        pltpu.VMEM((_CHUNK,), jnp.float32),
        pltpu.VMEM((_L,), jnp.float32),
        pltpu.SemaphoreType.DMA,
    ],
)(_sc_gather_body)


def kernel(x, target):
    x3 = x.reshape(_TOTAL_VECS, 8, 128)
    xflat = x.reshape(_R * _V)
    tgt = target.reshape(_R).astype(jnp.int32)

    sum_x = _tc_sum(x3)[0, 0]
    partials = _sc_gather(xflat, tgt)
    sum_gather = jnp.sum(partials)

    fill = jnp.float32(_FILL)
    conf_m_fill = jnp.float32(_CONF - _FILL)
    return jnp.float32(_TOTAL_CONST) - fill * sum_x - conf_m_fill * sum_gather


# speedup vs baseline: 5.7732x; 5.7732x over previous
import math

import jax
import jax.numpy as jnp
from jax.experimental import pallas as pl
from jax.experimental.pallas import tpu as pltpu

_B, _S, _V = 4, 2048, 16384
_SMOOTHING = 0.1
_CONF = 1.0 - _SMOOTHING
_R = _B * _S
_FILL = _SMOOTHING / (_V - 1)

_ROW_CONST = (_V - 1) * _FILL * math.log(_FILL) + _CONF * math.log(_CONF)
_TOTAL_CONST = _R * _ROW_CONST

_BLK_ROWS = 256
_NBLK = _R // _BLK_ROWS


def _body(x_ref, tgt_ref, sum_ref, gat_ref):
    i = pl.program_id(0)

    @pl.when(i == 0)
    def _init():
        sum_ref[0, 0] = 0.0
        gat_ref[0, 0] = 0.0

    x = x_ref[...]
    cols = jax.lax.broadcasted_iota(jnp.int32, (_BLK_ROWS, _V), 1)
    mask = cols == tgt_ref[0]
    sum_ref[0, 0] += jnp.sum(x)
    gat_ref[0, 0] += jnp.sum(jnp.where(mask, x, 0.0))


_fused = pl.pallas_call(
    _body,
    grid=(_NBLK,),
    in_specs=[
        pl.BlockSpec((_BLK_ROWS, _V), lambda i: (i, 0)),
        pl.BlockSpec((1, _BLK_ROWS, 1), lambda i: (i, 0, 0)),
    ],
    out_specs=[
        pl.BlockSpec(memory_space=pltpu.SMEM),
        pl.BlockSpec(memory_space=pltpu.SMEM),
    ],
    out_shape=[
        jax.ShapeDtypeStruct((1, 1), jnp.float32),
        jax.ShapeDtypeStruct((1, 1), jnp.float32),
    ],
)


def kernel(x, target):
    x2 = x.reshape(_R, _V)
    tgt = target.reshape(_NBLK, _BLK_ROWS, 1).astype(jnp.int32)

    sums, gats = _fused(x2, tgt)
    sum_x = sums[0, 0]
    sum_gather = gats[0, 0]

    fill = jnp.float32(_FILL)
    conf_m_fill = jnp.float32(_CONF - _FILL)
    return jnp.float32(_TOTAL_CONST) - fill * sum_x - conf_m_fill * sum_gather
